# baseline (speedup 1.0000x reference)
import jax
import jax.numpy as jnp
from jax.experimental import pallas as pl
from jax.experimental.pallas import tpu as pltpu

_D = 128


def _fused(x_ref, p_ref, o_ref):
    x = x_ref[...]
    wz = p_ref[0:_D, :]
    wh = p_ref[_D:2 * _D, :]
    wl = p_ref[2 * _D:3 * _D, :]
    t = p_ref[3 * _D:3 * _D + 8, :]
    bz = t[0:1, :]
    bh = t[1:2, :]
    blin = t[2:3, 0:1]
    z = jax.nn.sigmoid(jnp.dot(x, wz, preferred_element_type=jnp.float32) + bz)
    ht = jnp.tanh(jnp.dot(x, wh, preferred_element_type=jnp.float32) + bh)
    g = jax.nn.relu((1.0 - z) * ht)
    o_ref[...] = (
        jnp.dot(g, wl, preferred_element_type=jnp.float32)[:, 0:1] + blin
    )


def kernel(x, edge_index, edge_weight, W_xz, b_xz, W_hz, b_hz, W_xr, b_xr,
           W_hr, b_hr, W_xh, b_xh, W_hh, b_hh, W_lin, b_lin):
    n = x.shape[0]
    wlmat = jnp.pad(W_lin, ((0, 0), (0, _D - 1)))
    bz = (b_xz + b_hz).reshape(1, _D)
    bh = (b_xh + b_hh).reshape(1, _D)
    blin = jnp.broadcast_to(b_lin.reshape(1, 1), (1, _D))
    params = jnp.concatenate(
        [W_xz, W_xh, wlmat, bz, bh, blin, jnp.zeros((5, _D), jnp.float32)],
        axis=0,
    )

    vmem = pl.BlockSpec(memory_space=pltpu.MemorySpace.VMEM)
    return pl.pallas_call(
        _fused,
        in_specs=[vmem, vmem],
        out_specs=vmem,
        out_shape=jax.ShapeDtypeStruct((n, 1), x.dtype),
    )(x, params)


# speedup vs baseline: 1.0717x; 1.0717x over previous
import jax
import jax.numpy as jnp
from jax.experimental import pallas as pl
from jax.experimental.pallas import tpu as pltpu

_D = 128
_CH = 5000
_NCH = 2


def _fused(x_h, wz_ref, wh_ref, bxz_ref, bhz_ref, bxh_ref, bhh_ref,
           wlin_ref, blin_ref, o_ref, xb, sem):
    xcp = [
        pltpu.make_async_copy(
            x_h.at[pl.ds(i * _CH, _CH), :], xb.at[i], sem.at[i]
        )
        for i in range(_NCH)
    ]
    for c in xcp:
        c.start()

    wz = wz_ref[...]
    wh = wh_ref[...]
    bz = bxz_ref[...] + bhz_ref[...]
    bh = bxh_ref[...] + bhh_ref[...]
    wlin = wlin_ref[...]
    blin = blin_ref[...]

    for i in range(_NCH):
        xcp[i].wait()
        x = xb[i]
        z = jax.nn.sigmoid(
            jnp.dot(x, wz, preferred_element_type=jnp.float32) + bz
        )
        ht = jnp.tanh(
            jnp.dot(x, wh, preferred_element_type=jnp.float32) + bh
        )
        g = jax.nn.relu((1.0 - z) * ht)
        o_ref[pl.ds(i * _CH, _CH), :] = (
            jnp.dot(g, wlin, preferred_element_type=jnp.float32) + blin
        )


def kernel(x, edge_index, edge_weight, W_xz, b_xz, W_hz, b_hz, W_xr, b_xr,
           W_hr, b_hr, W_xh, b_xh, W_hh, b_hh, W_lin, b_lin):
    n = x.shape[0]
    vmem = pl.BlockSpec(memory_space=pltpu.MemorySpace.VMEM)
    hbm = pl.BlockSpec(memory_space=pltpu.MemorySpace.HBM)
    return pl.pallas_call(
        _fused,
        in_specs=[hbm] + [vmem] * 8,
        out_specs=vmem,
        out_shape=jax.ShapeDtypeStruct((n, 1), x.dtype),
        scratch_shapes=[
            pltpu.VMEM((_NCH, _CH, _D), jnp.float32),
            pltpu.SemaphoreType.DMA((_NCH,)),
        ],
    )(x, W_xz, W_xh, b_xz.reshape(1, _D), b_hz.reshape(1, _D),
      b_xh.reshape(1, _D), b_hh.reshape(1, _D), W_lin, b_lin.reshape(1, 1))


# speedup vs baseline: 1.2413x; 1.1582x over previous
import jax
import jax.numpy as jnp
from jax.experimental import pallas as pl
from jax.experimental.pallas import tpu as pltpu

_D = 128


def _fused(x_ref, wz_ref, wh_ref, bxz_ref, bhz_ref, bxh_ref, bhh_ref,
           wlin_ref, blin_ref, o_ref):
    x = x_ref[...].astype(jnp.bfloat16)
    wzn = (-(wz_ref[...])).astype(jnp.bfloat16)
    wh = wh_ref[...].astype(jnp.bfloat16)
    bzn = -(bxz_ref[...] + bhz_ref[...])
    bh = bxh_ref[...] + bhh_ref[...]
    zc = jax.nn.sigmoid(
        jnp.dot(x, wzn, preferred_element_type=jnp.float32) + bzn
    )
    ht = jnp.tanh(
        jnp.dot(x, wh, preferred_element_type=jnp.float32) + bh
    )
    g = jax.nn.relu(zc * ht)
    o_ref[...] = (
        jnp.dot(g, wlin_ref[...], preferred_element_type=jnp.float32)
        + blin_ref[...]
    )


def kernel(x, edge_index, edge_weight, W_xz, b_xz, W_hz, b_hz, W_xr, b_xr,
           W_hr, b_hr, W_xh, b_xh, W_hh, b_hh, W_lin, b_lin):
    n = x.shape[0]
    vmem = pl.BlockSpec(memory_space=pltpu.MemorySpace.VMEM)
    return pl.pallas_call(
        _fused,
        in_specs=[vmem] * 9,
        out_specs=vmem,
        out_shape=jax.ShapeDtypeStruct((n, 1), x.dtype),
    )(x, W_xz, W_xh, b_xz.reshape(1, _D), b_hz.reshape(1, _D),
      b_xh.reshape(1, _D), b_hh.reshape(1, _D), W_lin, b_lin.reshape(1, 1))


# speedup vs baseline: 1.3684x; 1.1024x over previous
import jax
import jax.numpy as jnp
from jax.experimental import pallas as pl
from jax.experimental.pallas import tpu as pltpu

_D = 128


def _fused(x_ref, wz_ref, wh_ref, bxz_ref, bhz_ref, bxh_ref, bhh_ref,
           wlin_ref, blin_ref, o_ref):
    x = x_ref[...]
    wz2 = wz_ref[...] * 0.5
    bz2 = (bxz_ref[...] + bhz_ref[...]) * 0.5
    wh = wh_ref[...]
    bh = bxh_ref[...] + bhh_ref[...]
    tz = jnp.tanh(jnp.dot(x, wz2, preferred_element_type=jnp.float32) + bz2)
    ht = jnp.tanh(jnp.dot(x, wh, preferred_element_type=jnp.float32) + bh)
    g = jax.nn.relu((1.0 - tz) * ht)
    o_ref[...] = (
        jnp.dot(g, wlin_ref[...] * 0.5, preferred_element_type=jnp.float32)
        + blin_ref[...]
    )


def kernel(x, edge_index, edge_weight, W_xz, b_xz, W_hz, b_hz, W_xr, b_xr,
           W_hr, b_hr, W_xh, b_xh, W_hh, b_hh, W_lin, b_lin):
    n = x.shape[0]
    vmem = pl.BlockSpec(memory_space=pltpu.MemorySpace.VMEM)
    return pl.pallas_call(
        _fused,
        in_specs=[vmem] * 9,
        out_specs=vmem,
        out_shape=jax.ShapeDtypeStruct((n, 1), x.dtype),
    )(x, W_xz, W_xh, b_xz.reshape(1, _D), b_hz.reshape(1, _D),
      b_xh.reshape(1, _D), b_hh.reshape(1, _D), W_lin, b_lin.reshape(1, 1))
